# Initial kernel scaffold; baseline (speedup 1.0000x reference)
import jax
import jax.numpy as jnp
from jax.experimental import pallas as pl


def kernel(x, edge_index, rel_emb, W_node, b_node, W_src, b_src, W_rel, W_res, b_res, res_w, w_cross, W_prop, b_prop):
    raise NotImplementedError("write your pallas kernel here")


# speedup vs baseline: 65.8453x; 65.8453x over previous
import functools
import numpy as np
import jax
import jax.numpy as jnp
from jax import lax
from jax.experimental import pallas as pl
from jax.experimental.pallas import tpu as pltpu
from jax.experimental.pallas import tpu_sc as plsc

N = 10000
E = 320000
D_IN = 128
H = 8
D_H = 16
D = H * D_H
R_IN = 64
ROW = D + 16

NB = 10
BN = N // NB

NTILES = 32
EPT = E // NTILES
CH = 80
NCH = EPT // CH
NPAD = 10240


def _consts():
    hh, dd = np.meshgrid(np.arange(H), np.arange(D_H), indexing="ij")
    src = (hh * D_H + dd).ravel()
    dstp = (dd * H + hh).ravel()
    P1 = np.zeros((D, D), np.float32)
    P1[src, dstp] = 1.0
    P2 = P1.T.copy()
    M8 = np.zeros((D, H), np.float32)
    M8[src, hh.ravel()] = 1.0
    Q = np.zeros((H, D), np.float32)
    Q[hh.ravel(), src] = 1.0
    S1 = np.zeros((2 * D, D), np.float32)
    S2 = np.zeros((2 * D, D), np.float32)
    S1[(hh * 2 * D_H + dd).ravel(), src] = 1.0
    S2[(hh * 2 * D_H + D_H + dd).ravel(), src] = 1.0
    return (jnp.asarray(P1), jnp.asarray(P2), jnp.asarray(M8),
            jnp.asarray(Q), jnp.asarray(S1), jnp.asarray(S2))


def _pre_body(x_ref, wn_ref, bn_ref, ws_ref, bs_ref, wr_ref, br_ref,
              rel_ref, wrel_ref, wprop_ref, bprop_ref,
              p1_ref, m8_ref, s1_ref, s2_ref,
              srct_ref, dstt_ref, res_ref, msrc_ref, relout_ref):
    i = pl.program_id(0)
    x = x_ref[...]
    h = jnp.dot(x, wn_ref[...], preferred_element_type=jnp.float32) + bn_ref[...]
    s = jnp.dot(h, ws_ref[...], preferred_element_type=jnp.float32) + bs_ref[...]
    r = jnp.dot(rel_ref[...], wrel_ref[...], preferred_element_type=jnp.float32)
    rsrc = jnp.dot(r, s1_ref[...], preferred_element_type=jnp.float32)
    rdst = jnp.dot(r, s2_ref[...], preferred_element_type=jnp.float32)
    es8 = jnp.dot(s * rsrc, m8_ref[...], preferred_element_type=jnp.float32)
    ed8 = jnp.dot(h * rdst, m8_ref[...], preferred_element_type=jnp.float32)
    sT = jnp.dot(s, p1_ref[...], preferred_element_type=jnp.float32)
    srct_ref[:, :D] = sT
    srct_ref[:, D:D + 8] = es8
    srct_ref[:, D + 8:] = es8
    dstt_ref[:, :8] = ed8
    dstt_ref[:, 8:] = ed8
    res_ref[...] = (jnp.dot(x, wr_ref[...], preferred_element_type=jnp.float32)
                    + br_ref[...])
    bmax = jnp.max(es8, axis=0, keepdims=True)

    @pl.when(i == 0)
    def _():
        msrc_ref[...] = bmax
        relout_ref[...] = (jnp.dot(rel_ref[...], wprop_ref[...],
                                   preferred_element_type=jnp.float32)
                           + bprop_ref[...])

    @pl.when(i > 0)
    def _():
        msrc_ref[...] = jnp.maximum(msrc_ref[...], bmax)


def _pre(x, W_node, b_node, W_src, b_src, W_res, b_res,
         rel2, W_rel, W_prop, b_prop2, P1, M8, S1, S2):
    full = lambda shape: pl.BlockSpec(shape, lambda i: (0, 0))
    return pl.pallas_call(
        _pre_body,
        grid=(NB,),
        in_specs=[
            pl.BlockSpec((BN, D_IN), lambda i: (i, 0)),
            full((D_IN, D)), full((1, D)),
            full((D, D)), full((1, D)),
            full((D_IN, D)), full((1, D)),
            full((1, R_IN)), full((R_IN, 2 * D)),
            full((R_IN, D)), full((1, D)),
            full((D, D)), full((D, H)), full((2 * D, D)), full((2 * D, D)),
        ],
        out_specs=[
            pl.BlockSpec((BN, ROW), lambda i: (i, 0)),
            pl.BlockSpec((BN, 16), lambda i: (i, 0)),
            pl.BlockSpec((BN, D), lambda i: (i, 0)),
            full((1, 8)),
            full((1, D)),
        ],
        out_shape=[
            jax.ShapeDtypeStruct((N, ROW), jnp.float32),
            jax.ShapeDtypeStruct((N, 16), jnp.float32),
            jax.ShapeDtypeStruct((N, D), jnp.float32),
            jax.ShapeDtypeStruct((1, 8), jnp.float32),
            jax.ShapeDtypeStruct((1, D), jnp.float32),
        ],
    )(x, W_node, b_node, W_src, b_src, W_res, b_res,
      rel2, W_rel, W_prop, b_prop2, P1, M8, S1, S2)


def _edge_body(srcid_hbm, dstid_hbm, srct_hbm, dstt_hbm, msrc_hbm,
               out_hbm,
               agg_sh, sidx, didx, srcbuf, dstbuf, msgbuf, msrc_v,
               sem_s, sem_d):
    c = lax.axis_index("c")
    s = lax.axis_index("s")
    wid = c * 16 + s
    pltpu.sync_copy(msrc_hbm, msrc_v)
    mreg = msrc_v[...]
    zero = jnp.zeros((16,), jnp.float32)

    def _zrow(e, _):
        for v in range(ROW // 16):
            msgbuf[e, pl.ds(v * 16, 16)] = zero
        return 0

    lax.fori_loop(0, CH, _zrow, 0)
    rows0 = s * (NPAD // 16)
    for k in range(NPAD // 16 // CH):
        pltpu.sync_copy(msgbuf, agg_sh.at[pl.ds(rows0 + k * CH, CH)])
    plsc.subcore_barrier()

    def _chunk(ci, _):
        base = wid * EPT + ci * CH
        pltpu.sync_copy(srcid_hbm.at[pl.ds(base, CH)], sidx)
        pltpu.sync_copy(dstid_hbm.at[pl.ds(base, CH)], didx.at[0])
        cp_s = pltpu.async_copy(srct_hbm.at[sidx], srcbuf, sem_s)
        cp_d = pltpu.async_copy(dstt_hbm.at[didx.at[0]], dstbuf, sem_d)
        cp_s.wait()
        cp_d.wait()

        def _edge(e, _):
            ed = dstbuf[e, :]
            es = srcbuf[e, pl.ds(D, 16)]
            t = es + ed
            lk = jnp.where(t > 0, t, t * 0.2)
            u = mreg + ed
            lb = jnp.where(u > 0, u, u * 0.2)
            ex = jnp.exp(lk - lb)
            for v in range(8):
                msgbuf[e, pl.ds(v * 16, 16)] = (
                    srcbuf[e, pl.ds(v * 16, 16)] * ex)
            msgbuf[e, pl.ds(D, 16)] = ex
            return 0

        lax.fori_loop(0, CH, _edge, 0)
        pltpu.sync_copy(msgbuf, agg_sh.at[didx.at[0]], add=True)
        return 0

    lax.fori_loop(0, NCH, _chunk, 0)
    plsc.subcore_barrier()
    pltpu.sync_copy(agg_sh.at[pl.ds(rows0, NPAD // 16)],
                    out_hbm.at[c].at[pl.ds(rows0, NPAD // 16)])


def _edge(src_ids, dst_ids, SRCT, DSTT, MSRC16):
    mesh = plsc.VectorSubcoreMesh(core_axis_name="c", subcore_axis_name="s")
    k = pl.kernel(
        _edge_body,
        out_type=jax.ShapeDtypeStruct((2, NPAD, ROW), jnp.float32),
        mesh=mesh,
        compiler_params=pltpu.CompilerParams(use_tc_tiling_on_sc=False),
        scratch_types=[
            pltpu.VMEM_SHARED((NPAD, ROW), jnp.float32),
            pltpu.VMEM((CH,), jnp.int32),
            pltpu.VMEM((1, CH), jnp.int32),
            pltpu.VMEM((CH, ROW), jnp.float32),
            pltpu.VMEM((CH, 16), jnp.float32),
            pltpu.VMEM((CH, ROW), jnp.float32),
            pltpu.VMEM((16,), jnp.float32),
            pltpu.SemaphoreType.DMA,
            pltpu.SemaphoreType.DMA,
        ],
    )
    return k(src_ids, dst_ids, SRCT, DSTT, MSRC16)


def _post_body(a0_ref, a1_ref, res_ref, p2_ref, q_ref, rw_ref, out_ref):
    t = a0_ref[...] + a1_ref[...]
    aggT = t[:, :D]
    den8 = t[:, D:D + 8]
    unnorm = jnp.dot(aggT, p2_ref[...], preferred_element_type=jnp.float32)
    denx = jnp.dot(den8, q_ref[...], preferred_element_type=jnp.float32)
    agg = unnorm / (denx + 1e-16)
    outv = jnp.maximum(agg, 0.0)
    a = 1.0 / (1.0 + jnp.exp(-rw_ref[0, 0]))
    out_ref[...] = outv * a + res_ref[...] * (1.0 - a)


def _post(A0, A1, RES, P2, Q, rw2):
    full = lambda shape: pl.BlockSpec(shape, lambda i: (0, 0))
    return pl.pallas_call(
        _post_body,
        grid=(NB,),
        in_specs=[
            pl.BlockSpec((BN, ROW), lambda i: (i, 0)),
            pl.BlockSpec((BN, ROW), lambda i: (i, 0)),
            pl.BlockSpec((BN, D), lambda i: (i, 0)),
            full((D, D)), full((H, D)), full((1, 1)),
        ],
        out_specs=pl.BlockSpec((BN, D), lambda i: (i, 0)),
        out_shape=jax.ShapeDtypeStruct((N, D), jnp.float32),
    )(A0, A1, RES, P2, Q, rw2)


@jax.jit
def kernel(x, edge_index, rel_emb, W_node, b_node, W_src, b_src, W_rel,
           W_res, b_res, res_w, w_cross, W_prop, b_prop):
    P1, P2, M8, Q, S1, S2 = _consts()
    rel2 = rel_emb.reshape(1, R_IN)
    SRCT, DSTT, RES, MSRC, RELOUT = _pre(
        x, W_node, b_node.reshape(1, D), W_src, b_src.reshape(1, D),
        W_res, b_res.reshape(1, D), rel2, W_rel, W_prop,
        b_prop.reshape(1, D), P1, M8, S1, S2)
    MSRC16 = jnp.concatenate([MSRC, MSRC], axis=1).reshape(16)
    src_ids = edge_index[0]
    dst_ids = edge_index[1]
    AGG = _edge(src_ids, dst_ids, SRCT, DSTT, MSRC16)
    crossed = _post(AGG[0, :N], AGG[1, :N], RES, P2, Q, res_w.reshape(1, 1))
    return crossed, RELOUT.reshape(D)


# speedup vs baseline: 83.1623x; 1.2630x over previous
import functools
import numpy as np
import jax
import jax.numpy as jnp
from jax import lax
from jax.experimental import pallas as pl
from jax.experimental.pallas import tpu as pltpu
from jax.experimental.pallas import tpu_sc as plsc

N = 10000
E = 320000
D_IN = 128
H = 8
D_H = 16
D = H * D_H
R_IN = 64
ROW = D + 16

NB = 10
BN = N // NB

NTILES = 32
EPT = E // NTILES
CH = 80
NCH = EPT // CH
NPAD = 10240


def _consts():
    hh, dd = np.meshgrid(np.arange(H), np.arange(D_H), indexing="ij")
    src = (hh * D_H + dd).ravel()
    dstp = (dd * H + hh).ravel()
    P1 = np.zeros((D, D), np.float32)
    P1[src, dstp] = 1.0
    P2 = P1.T.copy()
    M8 = np.zeros((D, H), np.float32)
    M8[src, hh.ravel()] = 1.0
    Q = np.zeros((H, D), np.float32)
    Q[hh.ravel(), src] = 1.0
    S1 = np.zeros((2 * D, D), np.float32)
    S2 = np.zeros((2 * D, D), np.float32)
    S1[(hh * 2 * D_H + dd).ravel(), src] = 1.0
    S2[(hh * 2 * D_H + D_H + dd).ravel(), src] = 1.0
    return (jnp.asarray(P1), jnp.asarray(P2), jnp.asarray(M8),
            jnp.asarray(Q), jnp.asarray(S1), jnp.asarray(S2))


def _pre_body(x_ref, wn_ref, bn_ref, ws_ref, bs_ref, wr_ref, br_ref,
              rel_ref, wrel_ref, wprop_ref, bprop_ref,
              p1_ref, m8_ref, s1_ref, s2_ref,
              srct_ref, dstt_ref, res_ref, msrc_ref, relout_ref):
    i = pl.program_id(0)
    x = x_ref[...]
    h = jnp.dot(x, wn_ref[...], preferred_element_type=jnp.float32) + bn_ref[...]
    s = jnp.dot(h, ws_ref[...], preferred_element_type=jnp.float32) + bs_ref[...]
    r = jnp.dot(rel_ref[...], wrel_ref[...], preferred_element_type=jnp.float32)
    rsrc = jnp.dot(r, s1_ref[...], preferred_element_type=jnp.float32)
    rdst = jnp.dot(r, s2_ref[...], preferred_element_type=jnp.float32)
    es8 = jnp.dot(s * rsrc, m8_ref[...], preferred_element_type=jnp.float32)
    ed8 = jnp.dot(h * rdst, m8_ref[...], preferred_element_type=jnp.float32)
    sT = jnp.dot(s, p1_ref[...], preferred_element_type=jnp.float32)
    srct_ref[:, :D] = sT
    srct_ref[:, D:D + 8] = es8
    srct_ref[:, D + 8:] = es8
    dstt_ref[:, :8] = ed8
    dstt_ref[:, 8:] = ed8
    res_ref[...] = (jnp.dot(x, wr_ref[...], preferred_element_type=jnp.float32)
                    + br_ref[...])
    bmax = jnp.max(es8, axis=0, keepdims=True)

    @pl.when(i == 0)
    def _():
        msrc_ref[...] = bmax
        relout_ref[...] = (jnp.dot(rel_ref[...], wprop_ref[...],
                                   preferred_element_type=jnp.float32)
                           + bprop_ref[...])

    @pl.when(i > 0)
    def _():
        msrc_ref[...] = jnp.maximum(msrc_ref[...], bmax)


def _pre(x, W_node, b_node, W_src, b_src, W_res, b_res,
         rel2, W_rel, W_prop, b_prop2, P1, M8, S1, S2):
    full = lambda shape: pl.BlockSpec(shape, lambda i: (0, 0))
    return pl.pallas_call(
        _pre_body,
        grid=(NB,),
        in_specs=[
            pl.BlockSpec((BN, D_IN), lambda i: (i, 0)),
            full((D_IN, D)), full((1, D)),
            full((D, D)), full((1, D)),
            full((D_IN, D)), full((1, D)),
            full((1, R_IN)), full((R_IN, 2 * D)),
            full((R_IN, D)), full((1, D)),
            full((D, D)), full((D, H)), full((2 * D, D)), full((2 * D, D)),
        ],
        out_specs=[
            pl.BlockSpec((BN, ROW), lambda i: (i, 0)),
            pl.BlockSpec((BN, 16), lambda i: (i, 0)),
            pl.BlockSpec((BN, D), lambda i: (i, 0)),
            full((1, 8)),
            full((1, D)),
        ],
        out_shape=[
            jax.ShapeDtypeStruct((N, ROW), jnp.float32),
            jax.ShapeDtypeStruct((N, 16), jnp.float32),
            jax.ShapeDtypeStruct((N, D), jnp.float32),
            jax.ShapeDtypeStruct((1, 8), jnp.float32),
            jax.ShapeDtypeStruct((1, D), jnp.float32),
        ],
    )(x, W_node, b_node, W_src, b_src, W_res, b_res,
      rel2, W_rel, W_prop, b_prop2, P1, M8, S1, S2)


def _edge_body(srcid_hbm, dstid_hbm, srct_hbm, dstt_hbm, msrc_hbm,
               out_hbm,
               agg_sh, sidx, didx, srcbuf, dstbuf, msgbuf, msrc_v,
               sem_s, sem_d):
    c = lax.axis_index("c")
    s = lax.axis_index("s")
    wid = c * 16 + s
    pltpu.sync_copy(msrc_hbm, msrc_v)
    mreg = msrc_v[...]
    zero = jnp.zeros((16,), jnp.float32)

    def _zrow(e, _):
        for v in range(ROW // 16):
            msgbuf[e, pl.ds(v * 16, 16)] = zero
        return 0

    lax.fori_loop(0, CH, _zrow, 0)
    rows0 = s * (NPAD // 16)
    for k in range(NPAD // 16 // CH):
        pltpu.sync_copy(msgbuf, agg_sh.at[pl.ds(rows0 + k * CH, CH)])
    plsc.subcore_barrier()

    def _chunk(ci, _):
        base = wid * EPT + ci * CH
        pltpu.sync_copy(srcid_hbm.at[pl.ds(base, CH)], sidx)
        pltpu.sync_copy(dstid_hbm.at[pl.ds(base, CH)], didx.at[0])
        cp_s = pltpu.async_copy(srct_hbm.at[sidx], srcbuf, sem_s)
        cp_d = pltpu.async_copy(dstt_hbm.at[didx.at[0]], dstbuf, sem_d)
        cp_s.wait()
        cp_d.wait()

        @plsc.parallel_loop(0, CH, 1, unroll=4)
        def _edge(e):
            ed = dstbuf[e, :]
            es = srcbuf[e, pl.ds(D, 16)]
            t = es + ed
            lk = jnp.where(t > 0, t, t * 0.2)
            u = mreg + ed
            lb = jnp.where(u > 0, u, u * 0.2)
            ex = jnp.exp(lk - lb)
            for v in range(8):
                msgbuf[e, pl.ds(v * 16, 16)] = (
                    srcbuf[e, pl.ds(v * 16, 16)] * ex)
            msgbuf[e, pl.ds(D, 16)] = ex
        pltpu.sync_copy(msgbuf, agg_sh.at[didx.at[0]], add=True)
        return 0

    lax.fori_loop(0, NCH, _chunk, 0)
    plsc.subcore_barrier()
    pltpu.sync_copy(agg_sh.at[pl.ds(rows0, NPAD // 16)],
                    out_hbm.at[c].at[pl.ds(rows0, NPAD // 16)])


def _edge(src_ids, dst_ids, SRCT, DSTT, MSRC16):
    mesh = plsc.VectorSubcoreMesh(core_axis_name="c", subcore_axis_name="s")
    k = pl.kernel(
        _edge_body,
        out_type=jax.ShapeDtypeStruct((2, NPAD, ROW), jnp.float32),
        mesh=mesh,
        compiler_params=pltpu.CompilerParams(use_tc_tiling_on_sc=False),
        scratch_types=[
            pltpu.VMEM_SHARED((NPAD, ROW), jnp.float32),
            pltpu.VMEM((CH,), jnp.int32),
            pltpu.VMEM((1, CH), jnp.int32),
            pltpu.VMEM((CH, ROW), jnp.float32),
            pltpu.VMEM((CH, 16), jnp.float32),
            pltpu.VMEM((CH, ROW), jnp.float32),
            pltpu.VMEM((16,), jnp.float32),
            pltpu.SemaphoreType.DMA,
            pltpu.SemaphoreType.DMA,
        ],
    )
    return k(src_ids, dst_ids, SRCT, DSTT, MSRC16)


def _post_body(a0_ref, a1_ref, res_ref, p2_ref, q_ref, rw_ref, out_ref):
    t = a0_ref[...] + a1_ref[...]
    aggT = t[:, :D]
    den8 = t[:, D:D + 8]
    unnorm = jnp.dot(aggT, p2_ref[...], preferred_element_type=jnp.float32)
    denx = jnp.dot(den8, q_ref[...], preferred_element_type=jnp.float32)
    agg = unnorm / (denx + 1e-16)
    outv = jnp.maximum(agg, 0.0)
    a = 1.0 / (1.0 + jnp.exp(-rw_ref[0, 0]))
    out_ref[...] = outv * a + res_ref[...] * (1.0 - a)


def _post(A0, A1, RES, P2, Q, rw2):
    full = lambda shape: pl.BlockSpec(shape, lambda i: (0, 0))
    return pl.pallas_call(
        _post_body,
        grid=(NB,),
        in_specs=[
            pl.BlockSpec((BN, ROW), lambda i: (i, 0)),
            pl.BlockSpec((BN, ROW), lambda i: (i, 0)),
            pl.BlockSpec((BN, D), lambda i: (i, 0)),
            full((D, D)), full((H, D)), full((1, 1)),
        ],
        out_specs=pl.BlockSpec((BN, D), lambda i: (i, 0)),
        out_shape=jax.ShapeDtypeStruct((N, D), jnp.float32),
    )(A0, A1, RES, P2, Q, rw2)


@jax.jit
def kernel(x, edge_index, rel_emb, W_node, b_node, W_src, b_src, W_rel,
           W_res, b_res, res_w, w_cross, W_prop, b_prop):
    P1, P2, M8, Q, S1, S2 = _consts()
    rel2 = rel_emb.reshape(1, R_IN)
    SRCT, DSTT, RES, MSRC, RELOUT = _pre(
        x, W_node, b_node.reshape(1, D), W_src, b_src.reshape(1, D),
        W_res, b_res.reshape(1, D), rel2, W_rel, W_prop,
        b_prop.reshape(1, D), P1, M8, S1, S2)
    MSRC16 = jnp.concatenate([MSRC, MSRC], axis=1).reshape(16)
    src_ids = edge_index[0]
    dst_ids = edge_index[1]
    AGG = _edge(src_ids, dst_ids, SRCT, DSTT, MSRC16)
    crossed = _post(AGG[0, :N], AGG[1, :N], RES, P2, Q, res_w.reshape(1, 1))
    return crossed, RELOUT.reshape(D)
